# baseline (speedup 1.0000x reference)
import functools

import jax
import jax.numpy as jnp
from jax.experimental import pallas as pl
from jax.experimental.pallas import tpu as pltpu

_CUTS = (0, 512, 640, 768)
_BB = 4
_SLOTS = 4


def _filter_kernel(B, C, H, W, ch_ref, x_ref, o_ref, xbuf, isem):
    NK = len(_CUTS) - 1
    NB = B // _BB
    ch = ch_ref[0]
    i = pl.program_id(0)

    def chunk_copy(ii, slot, k):
        lo, hi = _CUTS[k], _CUTS[k + 1]
        return pltpu.make_async_copy(
            x_ref.at[pl.ds(ii * _BB, _BB), :, :, pl.ds(lo, hi - lo)],
            xbuf.at[slot, :, :, :, pl.ds(lo, hi - lo)],
            isem.at[slot, k])

    def start_block(ii):
        for k in range(NK):
            @pl.when(_CUTS[k] < ch)
            def _(k=k):
                chunk_copy(ii, jax.lax.rem(ii, _SLOTS), k).start()

    @pl.when(i == 0)
    def _():
        for j in range(_SLOTS - 1):
            start_block(jnp.int32(j))

    @pl.when(i + _SLOTS - 1 < NB)
    def _():
        start_block(i + _SLOTS - 1)

    slot = jax.lax.rem(i, _SLOTS)
    for k in range(NK):
        @pl.when(_CUTS[k] < ch)
        def _(k=k):
            chunk_copy(i, slot, k).wait()

    c = jax.lax.broadcasted_iota(jnp.int32, (_BB, H, W, C), 3)
    o_ref[...] = jnp.where(c < ch, xbuf[slot], 0.0)


def kernel(x, channels):
    B, C, H, W = x.shape
    xt = jnp.transpose(x, (0, 2, 3, 1))
    ch = jnp.asarray(channels, jnp.int32).reshape(1)
    out = pl.pallas_call(
        functools.partial(_filter_kernel, B, C, H, W),
        grid_spec=pltpu.PrefetchScalarGridSpec(
            num_scalar_prefetch=1,
            grid=(B // _BB,),
            in_specs=[pl.BlockSpec(memory_space=pltpu.MemorySpace.HBM)],
            out_specs=pl.BlockSpec((_BB, H, W, C), lambda i, ch: (i, 0, 0, 0)),
            scratch_shapes=[
                pltpu.VMEM((_SLOTS, _BB, H, W, C), x.dtype),
                pltpu.SemaphoreType.DMA((_SLOTS, len(_CUTS) - 1)),
            ],
        ),
        out_shape=jax.ShapeDtypeStruct((B, H, W, C), x.dtype),
    )(ch, xt)
    return jnp.transpose(out, (0, 3, 1, 2))


# speedup vs baseline: 1.0099x; 1.0099x over previous
import functools

import jax
import jax.numpy as jnp
from jax.experimental import pallas as pl
from jax.experimental.pallas import tpu as pltpu

_CUTS = (0, 512, 640, 768)
_BB = 8
_SLOTS = 2


def _filter_kernel(B, C, H, W, ch_ref, x_ref, o_ref, xbuf, isem):
    NK = len(_CUTS) - 1
    NB = B // _BB
    ch = ch_ref[0]
    i = pl.program_id(0)

    def chunk_copy(ii, slot, k):
        lo, hi = _CUTS[k], _CUTS[k + 1]
        return pltpu.make_async_copy(
            x_ref.at[pl.ds(ii * _BB, _BB), :, :, pl.ds(lo, hi - lo)],
            xbuf.at[slot, :, :, :, pl.ds(lo, hi - lo)],
            isem.at[slot, k])

    def start_block(ii):
        for k in range(NK):
            @pl.when(_CUTS[k] < ch)
            def _(k=k):
                chunk_copy(ii, jax.lax.rem(ii, _SLOTS), k).start()

    @pl.when(i == 0)
    def _():
        for j in range(_SLOTS - 1):
            start_block(jnp.int32(j))

    @pl.when(i + _SLOTS - 1 < NB)
    def _():
        start_block(i + _SLOTS - 1)

    slot = jax.lax.rem(i, _SLOTS)
    for k in range(NK):
        @pl.when(_CUTS[k] < ch)
        def _(k=k):
            chunk_copy(i, slot, k).wait()

    c = jax.lax.broadcasted_iota(jnp.int32, (_BB, H, W, C), 3)
    o_ref[...] = jnp.where(c < ch, xbuf[slot], 0.0)


def kernel(x, channels):
    B, C, H, W = x.shape
    xt = jnp.transpose(x, (0, 2, 3, 1))
    ch = jnp.asarray(channels, jnp.int32).reshape(1)
    out = pl.pallas_call(
        functools.partial(_filter_kernel, B, C, H, W),
        grid_spec=pltpu.PrefetchScalarGridSpec(
            num_scalar_prefetch=1,
            grid=(B // _BB,),
            in_specs=[pl.BlockSpec(memory_space=pltpu.MemorySpace.HBM)],
            out_specs=pl.BlockSpec((_BB, H, W, C), lambda i, ch: (i, 0, 0, 0)),
            scratch_shapes=[
                pltpu.VMEM((_SLOTS, _BB, H, W, C), x.dtype),
                pltpu.SemaphoreType.DMA((_SLOTS, len(_CUTS) - 1)),
            ],
        ),
        out_shape=jax.ShapeDtypeStruct((B, H, W, C), x.dtype),
    )(ch, xt)
    return jnp.transpose(out, (0, 3, 1, 2))
